# baseline (speedup 1.0000x reference)
import functools

import jax
import jax.numpy as jnp
from jax import lax
from jax.experimental import pallas as pl
from jax.experimental.pallas import tpu as pltpu
from jax.experimental.pallas import tpu_sc as plsc

NODE_NUM = 1000000
EMB_DIM = 32
BATCH = 4096
HIST = 50

NC = 2
NS = 16
NW = NC * NS
BPW = BATCH // NW

INFLIGHT = 16


def _sc_body(table_hbm, data_hbm, out_hbm, idx_v, cols_v, acc_v, out_v, sem):
  wid = lax.axis_index("s") * NC + lax.axis_index("c")
  base = wid * BPW

  pltpu.sync_copy(data_hbm.at[pl.ds(base, BPW)], idx_v)

  zeros = jnp.zeros((16,), jnp.float32)

  def zbody(b, c):
    acc_v[b, pl.ds(0, 16)] = zeros
    acc_v[b, pl.ds(16, 16)] = zeros
    return c
  lax.fori_loop(0, BPW, zbody, 0, unroll=8)

  lanes = lax.iota(jnp.int32, 16)

  def tbody(j, c):
    jv = jnp.full((16,), j, jnp.int32)
    for b0 in range(0, BPW, 16):
      vals = plsc.load_gather(idx_v, [lanes + b0, jv])
      cols_v[j, pl.ds(b0, 16)] = vals
    return c
  lax.fori_loop(0, HIST, tbody, 0)

  def gather_add(j):
    pltpu.async_copy(table_hbm.at[cols_v.at[j]], acc_v, sem, add=True)

  def drain_one():
    pltpu.make_async_copy(table_hbm.at[cols_v.at[0]], acc_v, sem).wait()

  for j in range(INFLIGHT):
    gather_add(j)
  for j in range(INFLIGHT, HIST):
    drain_one()
    gather_add(j)
  for _ in range(INFLIGHT):
    drain_one()

  scale = jnp.float32(1.0 / HIST)

  def finish(b, c):
    out_v[b, pl.ds(0, 16)] = acc_v[b, pl.ds(0, 16)] * scale
    out_v[b, pl.ds(16, 16)] = acc_v[b, pl.ds(16, 16)] * scale
    return c
  lax.fori_loop(0, BPW, finish, 0, unroll=8)

  pltpu.sync_copy(out_v, out_hbm.at[pl.ds(base, BPW)])


@jax.jit
def _graph_encode(data, table):
  mesh = plsc.VectorSubcoreMesh(
      core_axis_name="c", subcore_axis_name="s", num_cores=NC, num_subcores=NS)
  k = pl.kernel(
      _sc_body,
      out_type=jax.ShapeDtypeStruct((BATCH, EMB_DIM), jnp.float32),
      mesh=mesh,
      scratch_types=[
          pltpu.VMEM((BPW, HIST), jnp.int32),
          pltpu.VMEM((HIST, BPW), jnp.int32),
          pltpu.VMEM((BPW, EMB_DIM), jnp.float32),
          pltpu.VMEM((BPW, EMB_DIM), jnp.float32),
          pltpu.SemaphoreType.DMA,
      ],
      compiler_params=pltpu.CompilerParams(
          use_tc_tiling_on_sc=False, needs_layout_passes=False),
  )
  return k(table, data)


def kernel(data, table):
  return _graph_encode(data, table)
